# baseline (speedup 1.0000x reference)
import jax
import jax.numpy as jnp
from jax.experimental import pallas as pl
from jax.experimental.pallas import tpu as pltpu

N_J = 32
N_M = 16
B = 8
N = B * N_J * N_M
INPUT_DIM = 128
HIDDEN = 64
H_ACT = 32
H_CRI = 32
BLK = 1024
NB = N // BLK
BC = B * N_J

_HI = jax.lax.Precision.HIGHEST


def _bn(h):
    m = jnp.mean(h, axis=0, keepdims=True)
    v = jnp.mean((h - m) * (h - m), axis=0, keepdims=True)
    return (h - m) / jnp.sqrt(v + 1e-5)


def _dot(a, b, prec=None):
    return jnp.dot(a, b, preferred_element_type=jnp.float32, precision=prec)


def _fused_kernel(adj_ref, x_ref, gp_ref, cc_ref, bx_ref, mk_ref,
                  g0w1_ref, g0w2_ref, g1w1_ref, g1w2_ref,
                  aw1_ref, ab1_ref, aw2_ref, ab2_ref,
                  crw1_ref, crb1_ref, crw2_ref, crb2_ref,
                  ccw1_ref, ccb1_ref, ccw2_ref, ccb2_ref,
                  pi_ref, v_ref, vc_ref,
                  p0_scr, p1_scr, y_scr):
    ph = pl.program_id(0)
    i = pl.program_id(1)

    @pl.when(ph == 0)
    def _stream_layer0():
        p0_scr[pl.ds(i * BLK, BLK), :] = _dot(adj_ref[...], x_ref[...])

    @pl.when((ph == 1) & (i == 0))
    def _layer_transition():
        hid = jax.nn.relu(_bn(_dot(p0_scr[...], g0w1_ref[...])))
        rep = _dot(hid, g0w2_ref[...])
        y_scr[...] = jax.nn.relu(_bn(rep))

    @pl.when(ph == 1)
    def _stream_layer1():
        p1_scr[pl.ds(i * BLK, BLK), :] = _dot(adj_ref[...], y_scr[...])

    @pl.when((ph == 1) & (i == NB - 1))
    def _epilogue():
        hid = jax.nn.relu(_bn(_dot(p1_scr[...], g1w1_ref[...])))
        rep = _dot(hid, g1w2_ref[...])
        h1 = jax.nn.relu(_bn(rep))

        hp = _dot(gp_ref[...], h1)

        cols = jax.lax.broadcasted_iota(jnp.int32, (BC, N), 1)
        oh = (cols == cc_ref[...]).astype(jnp.float32)
        cf = _dot(oh, h1, _HI)

        bcols = jax.lax.broadcasted_iota(jnp.int32, (BC, B), 1)
        ohb = (bcols == bx_ref[...]).astype(jnp.float32)
        hpr = _dot(ohb, hp, _HI)

        aw1 = aw1_ref[...]
        t = jnp.tanh(_dot(cf, aw1[:HIDDEN, :])
                     + _dot(hpr, aw1[HIDDEN:, :]) + ab1_ref[...])
        s = _dot(t, aw2_ref[...]) + ab2_ref[...]
        s = jnp.where(mk_ref[...] != 0.0, -1e30, s)

        e = jnp.exp(s)
        den = jax.lax.dot_general(ohb, e, (((0,), (0,)), ((), ())),
                                  preferred_element_type=jnp.float32,
                                  precision=_HI)
        rden = _dot(ohb, den, _HI)
        pi_ref[...] = e / rden

        v_ref[...] = _dot(jnp.tanh(_dot(hp, crw1_ref[...]) + crb1_ref[...]),
                          crw2_ref[...]) + crb2_ref[...]
        vc_ref[...] = _dot(jnp.tanh(_dot(hp, ccw1_ref[...]) + ccb1_ref[...]),
                           ccw2_ref[...]) + ccb2_ref[...]


@jax.jit
def _run(x, graph_pool, adj, cand_cols, bidx, mask_col,
         g0w1, g0w2, g1w1, g1w2, aw1, ab1, aw2, ab2,
         crw1, crb1, crw2, crb2, ccw1, ccb1, ccw2, ccb2):
    res = lambda shp: pl.BlockSpec(shp, lambda p, i: (0,) * len(shp))
    pi_flat, v, v_c = pl.pallas_call(
        _fused_kernel,
        grid=(2, NB),
        in_specs=[
            pl.BlockSpec((BLK, N), lambda p, i: (i, 0)),
            res((N, INPUT_DIM)),
            res((B, N)),
            res((BC, 1)),
            res((BC, 1)),
            res((BC, 1)),
            res((INPUT_DIM, HIDDEN)), res((HIDDEN, HIDDEN)),
            res((HIDDEN, HIDDEN)), res((HIDDEN, HIDDEN)),
            res((2 * HIDDEN, H_ACT)), res((1, H_ACT)),
            res((H_ACT, 1)), res((1, 1)),
            res((HIDDEN, H_CRI)), res((1, H_CRI)),
            res((H_CRI, 1)), res((1, 1)),
            res((HIDDEN, H_CRI)), res((1, H_CRI)),
            res((H_CRI, 1)), res((1, 1)),
        ],
        out_specs=[res((BC, 1)), res((B, 1)), res((B, 1))],
        out_shape=[
            jax.ShapeDtypeStruct((BC, 1), jnp.float32),
            jax.ShapeDtypeStruct((B, 1), jnp.float32),
            jax.ShapeDtypeStruct((B, 1), jnp.float32),
        ],
        scratch_shapes=[
            pltpu.VMEM((N, INPUT_DIM), jnp.float32),
            pltpu.VMEM((N, HIDDEN), jnp.float32),
            pltpu.VMEM((N, HIDDEN), jnp.float32),
        ],
        compiler_params=pltpu.CompilerParams(
            dimension_semantics=("arbitrary", "arbitrary")),
    )(adj, x, graph_pool, cand_cols, bidx, mask_col,
      g0w1, g0w2, g1w1, g1w2, aw1, ab1, aw2, ab2,
      crw1, crb1, crw2, crb2, ccw1, ccb1, ccw2, ccb2)
    return pi_flat.reshape(B, N_J, 1), v, v_c


def kernel(x, graph_pool, padded_nei, adj, candidate, mask,
           g0w1, g0b1, g0w2, g0b2, g1w1, g1b1, g1w2, g1b2,
           aw1, ab1, aw2, ab2, crw1, crb1, crw2, crb2,
           ccw1, ccb1, ccw2, ccb2):
    del padded_nei, g0b1, g0b2, g1b1, g1b2
    boff = jnp.arange(B, dtype=jnp.int32)[:, None] * (N_J * N_M)
    cand_cols = (candidate.astype(jnp.int32) + boff).reshape(BC, 1)
    bidx = (jnp.arange(BC, dtype=jnp.int32) // N_J).reshape(BC, 1)
    mask_col = mask.astype(jnp.float32).reshape(BC, 1)
    return _run(x, graph_pool, adj, cand_cols, bidx, mask_col,
                g0w1, g0w2, g1w1, g1w2,
                aw1, ab1.reshape(1, H_ACT), aw2, ab2.reshape(1, 1),
                crw1, crb1.reshape(1, H_CRI), crw2, crb2.reshape(1, 1),
                ccw1, ccb1.reshape(1, H_CRI), ccw2, ccb2.reshape(1, 1))


# speedup vs baseline: 1.0086x; 1.0086x over previous
import jax
import jax.numpy as jnp
from jax.experimental import pallas as pl
from jax.experimental.pallas import tpu as pltpu

N_J = 32
N_M = 16
B = 8
N = B * N_J * N_M
INPUT_DIM = 128
HIDDEN = 64
H_ACT = 32
H_CRI = 32
BLK = 512
NB = N // BLK
NH = NB // 2
BC = B * N_J

_HI = jax.lax.Precision.HIGHEST


def _bn(h):
    m = jnp.mean(h, axis=0, keepdims=True)
    v = jnp.mean((h - m) * (h - m), axis=0, keepdims=True)
    return (h - m) / jnp.sqrt(v + 1e-5)


def _dot(a, b, prec=None):
    return jnp.dot(a, b, preferred_element_type=jnp.float32, precision=prec)


def _fused_kernel(adja_ref, adjb_ref, x_ref, gp_ref, cc_ref, bx_ref, mk_ref,
                  g0w1_ref, g0w2_ref, g1w1_ref, g1w2_ref,
                  aw1_ref, ab1_ref, aw2_ref, ab2_ref,
                  crw1_ref, crb1_ref, crw2_ref, crb2_ref,
                  ccw1_ref, ccb1_ref, ccw2_ref, ccb2_ref,
                  pi_ref, v_ref, vc_ref,
                  p0_scr, p1_scr, y_scr):
    ph = pl.program_id(0)
    i = pl.program_id(1)

    @pl.when(ph == 0)
    def _stream_layer0():
        p0_scr[pl.ds(i * BLK, BLK), :] = _dot(adja_ref[...], x_ref[...])
        p0_scr[pl.ds((i + NH) * BLK, BLK), :] = _dot(adjb_ref[...], x_ref[...])

    @pl.when((ph == 1) & (i == 0))
    def _layer_transition():
        hid = jax.nn.relu(_bn(_dot(p0_scr[...], g0w1_ref[...])))
        rep = _dot(hid, g0w2_ref[...])
        y_scr[...] = jax.nn.relu(_bn(rep))

    @pl.when(ph == 1)
    def _stream_layer1():
        p1_scr[pl.ds(i * BLK, BLK), :] = _dot(adja_ref[...], y_scr[...])
        p1_scr[pl.ds((i + NH) * BLK, BLK), :] = _dot(adjb_ref[...], y_scr[...])

    @pl.when((ph == 1) & (i == NH - 1))
    def _epilogue():
        hid = jax.nn.relu(_bn(_dot(p1_scr[...], g1w1_ref[...])))
        rep = _dot(hid, g1w2_ref[...])
        h1 = jax.nn.relu(_bn(rep))

        hp = _dot(gp_ref[...], h1)

        cols = jax.lax.broadcasted_iota(jnp.int32, (BC, N), 1)
        oh = (cols == cc_ref[...]).astype(jnp.float32)
        cf = _dot(oh, h1, _HI)

        bcols = jax.lax.broadcasted_iota(jnp.int32, (BC, B), 1)
        ohb = (bcols == bx_ref[...]).astype(jnp.float32)
        hpr = _dot(ohb, hp, _HI)

        aw1 = aw1_ref[...]
        t = jnp.tanh(_dot(cf, aw1[:HIDDEN, :])
                     + _dot(hpr, aw1[HIDDEN:, :]) + ab1_ref[...])
        s = _dot(t, aw2_ref[...]) + ab2_ref[...]
        s = jnp.where(mk_ref[...] != 0.0, -1e30, s)

        e = jnp.exp(s)
        den = jax.lax.dot_general(ohb, e, (((0,), (0,)), ((), ())),
                                  preferred_element_type=jnp.float32,
                                  precision=_HI)
        rden = _dot(ohb, den, _HI)
        pi_ref[...] = e / rden

        v_ref[...] = _dot(jnp.tanh(_dot(hp, crw1_ref[...]) + crb1_ref[...]),
                          crw2_ref[...]) + crb2_ref[...]
        vc_ref[...] = _dot(jnp.tanh(_dot(hp, ccw1_ref[...]) + ccb1_ref[...]),
                           ccw2_ref[...]) + ccb2_ref[...]


@jax.jit
def _run(x, graph_pool, adj, cand_cols, bidx, mask_col,
         g0w1, g0w2, g1w1, g1w2, aw1, ab1, aw2, ab2,
         crw1, crb1, crw2, crb2, ccw1, ccb1, ccw2, ccb2):
    res = lambda shp: pl.BlockSpec(shp, lambda p, i: (0,) * len(shp))
    pi_flat, v, v_c = pl.pallas_call(
        _fused_kernel,
        grid=(2, NH),
        in_specs=[
            pl.BlockSpec((BLK, N), lambda p, i: (i, 0)),
            pl.BlockSpec((BLK, N), lambda p, i: (i + NH, 0)),
            res((N, INPUT_DIM)),
            res((B, N)),
            res((BC, 1)),
            res((BC, 1)),
            res((BC, 1)),
            res((INPUT_DIM, HIDDEN)), res((HIDDEN, HIDDEN)),
            res((HIDDEN, HIDDEN)), res((HIDDEN, HIDDEN)),
            res((2 * HIDDEN, H_ACT)), res((1, H_ACT)),
            res((H_ACT, 1)), res((1, 1)),
            res((HIDDEN, H_CRI)), res((1, H_CRI)),
            res((H_CRI, 1)), res((1, 1)),
            res((HIDDEN, H_CRI)), res((1, H_CRI)),
            res((H_CRI, 1)), res((1, 1)),
        ],
        out_specs=[res((BC, 1)), res((B, 1)), res((B, 1))],
        out_shape=[
            jax.ShapeDtypeStruct((BC, 1), jnp.float32),
            jax.ShapeDtypeStruct((B, 1), jnp.float32),
            jax.ShapeDtypeStruct((B, 1), jnp.float32),
        ],
        scratch_shapes=[
            pltpu.VMEM((N, INPUT_DIM), jnp.float32),
            pltpu.VMEM((N, HIDDEN), jnp.float32),
            pltpu.VMEM((N, HIDDEN), jnp.float32),
        ],
        compiler_params=pltpu.CompilerParams(
            dimension_semantics=("arbitrary", "arbitrary")),
    )(adj, adj, x, graph_pool, cand_cols, bidx, mask_col,
      g0w1, g0w2, g1w1, g1w2, aw1, ab1, aw2, ab2,
      crw1, crb1, crw2, crb2, ccw1, ccb1, ccw2, ccb2)
    return pi_flat.reshape(B, N_J, 1), v, v_c


def kernel(x, graph_pool, padded_nei, adj, candidate, mask,
           g0w1, g0b1, g0w2, g0b2, g1w1, g1b1, g1w2, g1b2,
           aw1, ab1, aw2, ab2, crw1, crb1, crw2, crb2,
           ccw1, ccb1, ccw2, ccb2):
    del padded_nei, g0b1, g0b2, g1b1, g1b2
    boff = jnp.arange(B, dtype=jnp.int32)[:, None] * (N_J * N_M)
    cand_cols = (candidate.astype(jnp.int32) + boff).reshape(BC, 1)
    bidx = (jnp.arange(BC, dtype=jnp.int32) // N_J).reshape(BC, 1)
    mask_col = mask.astype(jnp.float32).reshape(BC, 1)
    return _run(x, graph_pool, adj, cand_cols, bidx, mask_col,
                g0w1, g0w2, g1w1, g1w2,
                aw1, ab1.reshape(1, H_ACT), aw2, ab2.reshape(1, 1),
                crw1, crb1.reshape(1, H_CRI), crw2, crb2.reshape(1, 1),
                ccw1, ccb1.reshape(1, H_CRI), ccw2, ccb2.reshape(1, 1))


# speedup vs baseline: 1.0256x; 1.0169x over previous
import jax
import jax.numpy as jnp
from jax.experimental import pallas as pl
from jax.experimental.pallas import tpu as pltpu

N_J = 32
N_M = 16
B = 8
N = B * N_J * N_M
INPUT_DIM = 128
HIDDEN = 64
H_ACT = 32
H_CRI = 32
BLK = 256
NB = N // BLK
BC = B * N_J

_HI = jax.lax.Precision.HIGHEST


def _bn(h):
    m = jnp.mean(h, axis=0, keepdims=True)
    v = jnp.mean((h - m) * (h - m), axis=0, keepdims=True)
    return (h - m) / jnp.sqrt(v + 1e-5)


def _dot(a, b, prec=None):
    return jnp.dot(a, b, preferred_element_type=jnp.float32, precision=prec)


def _fused_kernel(adj_ref, x_ref, gp_ref, cc_ref, bx_ref, mk_ref,
                  g0w1_ref, g0w2_ref, g1w1_ref, g1w2_ref,
                  aw1_ref, ab1_ref, aw2_ref, ab2_ref,
                  crw1_ref, crb1_ref, crw2_ref, crb2_ref,
                  ccw1_ref, ccb1_ref, ccw2_ref, ccb2_ref,
                  pi_ref, v_ref, vc_ref,
                  adjbf_scr, p0_scr, p1_scr, y_scr):
    ph = pl.program_id(0)
    i = pl.program_id(1)

    @pl.when(ph == 0)
    def _stream_layer0():
        blk = adj_ref[...]
        p0_scr[pl.ds(i * BLK, BLK), :] = _dot(blk, x_ref[...])
        adjbf_scr[pl.ds(i * BLK, BLK), :] = blk.astype(jnp.bfloat16)

    @pl.when((ph == 1) & (i == 0))
    def _layer_transition():
        hid = jax.nn.relu(_bn(_dot(p0_scr[...], g0w1_ref[...])))
        rep = _dot(hid, g0w2_ref[...])
        y_scr[...] = jax.nn.relu(_bn(rep)).astype(jnp.bfloat16)

    @pl.when(ph == 1)
    def _stream_layer1():
        p1_scr[pl.ds(i * BLK, BLK), :] = _dot(
            adjbf_scr[pl.ds(i * BLK, BLK), :], y_scr[...])

    @pl.when((ph == 1) & (i == NB - 1))
    def _epilogue():
        hid = jax.nn.relu(_bn(_dot(p1_scr[...], g1w1_ref[...])))
        rep = _dot(hid, g1w2_ref[...])
        h1 = jax.nn.relu(_bn(rep))

        hp = _dot(gp_ref[...], h1)

        cols = jax.lax.broadcasted_iota(jnp.int32, (BC, N), 1)
        oh = (cols == cc_ref[...]).astype(jnp.float32)
        cf = _dot(oh, h1, _HI)

        bcols = jax.lax.broadcasted_iota(jnp.int32, (BC, B), 1)
        ohb = (bcols == bx_ref[...]).astype(jnp.float32)
        hpr = _dot(ohb, hp, _HI)

        aw1 = aw1_ref[...]
        t = jnp.tanh(_dot(cf, aw1[:HIDDEN, :])
                     + _dot(hpr, aw1[HIDDEN:, :]) + ab1_ref[...])
        s = _dot(t, aw2_ref[...]) + ab2_ref[...]
        s = jnp.where(mk_ref[...] != 0.0, -1e30, s)

        e = jnp.exp(s)
        den = jax.lax.dot_general(ohb, e, (((0,), (0,)), ((), ())),
                                  preferred_element_type=jnp.float32,
                                  precision=_HI)
        rden = _dot(ohb, den, _HI)
        pi_ref[...] = e / rden

        v_ref[...] = _dot(jnp.tanh(_dot(hp, crw1_ref[...]) + crb1_ref[...]),
                          crw2_ref[...]) + crb2_ref[...]
        vc_ref[...] = _dot(jnp.tanh(_dot(hp, ccw1_ref[...]) + ccb1_ref[...]),
                           ccw2_ref[...]) + ccb2_ref[...]


@jax.jit
def _run(x, graph_pool, adj, cand_cols, bidx, mask_col,
         g0w1, g0w2, g1w1, g1w2, aw1, ab1, aw2, ab2,
         crw1, crb1, crw2, crb2, ccw1, ccb1, ccw2, ccb2):
    res = lambda shp: pl.BlockSpec(shp, lambda p, i: (0,) * len(shp))
    pi_flat, v, v_c = pl.pallas_call(
        _fused_kernel,
        grid=(2, NB),
        in_specs=[
            pl.BlockSpec((BLK, N), lambda p, i: (i * (1 - p), 0)),
            res((N, INPUT_DIM)),
            res((B, N)),
            res((BC, 1)),
            res((BC, 1)),
            res((BC, 1)),
            res((INPUT_DIM, HIDDEN)), res((HIDDEN, HIDDEN)),
            res((HIDDEN, HIDDEN)), res((HIDDEN, HIDDEN)),
            res((2 * HIDDEN, H_ACT)), res((1, H_ACT)),
            res((H_ACT, 1)), res((1, 1)),
            res((HIDDEN, H_CRI)), res((1, H_CRI)),
            res((H_CRI, 1)), res((1, 1)),
            res((HIDDEN, H_CRI)), res((1, H_CRI)),
            res((H_CRI, 1)), res((1, 1)),
        ],
        out_specs=[res((BC, 1)), res((B, 1)), res((B, 1))],
        out_shape=[
            jax.ShapeDtypeStruct((BC, 1), jnp.float32),
            jax.ShapeDtypeStruct((B, 1), jnp.float32),
            jax.ShapeDtypeStruct((B, 1), jnp.float32),
        ],
        scratch_shapes=[
            pltpu.VMEM((N, N), jnp.bfloat16),
            pltpu.VMEM((N, INPUT_DIM), jnp.float32),
            pltpu.VMEM((N, HIDDEN), jnp.float32),
            pltpu.VMEM((N, HIDDEN), jnp.bfloat16),
        ],
        compiler_params=pltpu.CompilerParams(
            dimension_semantics=("arbitrary", "arbitrary")),
    )(adj, x, graph_pool, cand_cols, bidx, mask_col,
      g0w1, g0w2, g1w1, g1w2, aw1, ab1, aw2, ab2,
      crw1, crb1, crw2, crb2, ccw1, ccb1, ccw2, ccb2)
    return pi_flat.reshape(B, N_J, 1), v, v_c


def kernel(x, graph_pool, padded_nei, adj, candidate, mask,
           g0w1, g0b1, g0w2, g0b2, g1w1, g1b1, g1w2, g1b2,
           aw1, ab1, aw2, ab2, crw1, crb1, crw2, crb2,
           ccw1, ccb1, ccw2, ccb2):
    del padded_nei, g0b1, g0b2, g1b1, g1b2
    boff = jnp.arange(B, dtype=jnp.int32)[:, None] * (N_J * N_M)
    cand_cols = (candidate.astype(jnp.int32) + boff).reshape(BC, 1)
    bidx = (jnp.arange(BC, dtype=jnp.int32) // N_J).reshape(BC, 1)
    mask_col = mask.astype(jnp.float32).reshape(BC, 1)
    return _run(x, graph_pool, adj, cand_cols, bidx, mask_col,
                g0w1, g0w2, g1w1, g1w2,
                aw1, ab1.reshape(1, H_ACT), aw2, ab2.reshape(1, 1),
                crw1, crb1.reshape(1, H_CRI), crw2, crb2.reshape(1, 1),
                ccw1, ccb1.reshape(1, H_CRI), ccw2, ccb2.reshape(1, 1))


# speedup vs baseline: 1.0635x; 1.0369x over previous
import jax
import jax.numpy as jnp
from jax.experimental import pallas as pl
from jax.experimental.pallas import tpu as pltpu

N_J = 32
N_M = 16
B = 8
N = B * N_J * N_M
INPUT_DIM = 128
HIDDEN = 64
H_ACT = 32
H_CRI = 32
BLK = 256
NB = N // BLK
BC = B * N_J

_HI = jax.lax.Precision.HIGHEST


def _bn(h):
    m = jnp.mean(h, axis=0, keepdims=True)
    v = jnp.mean((h - m) * (h - m), axis=0, keepdims=True)
    return (h - m) / jnp.sqrt(v + 1e-5)


def _dot(a, b, prec=None):
    return jnp.dot(a, b, preferred_element_type=jnp.float32, precision=prec)


def _fused_kernel(adj_ref, x_ref, gp_ref, cand_ref, mk_ref,
                  g0w1_ref, g0w2_ref, g1w1_ref, g1w2_ref,
                  aw1_ref, ab1_ref, aw2_ref, ab2_ref,
                  crw1_ref, crb1_ref, crw2_ref, crb2_ref,
                  ccw1_ref, ccb1_ref, ccw2_ref, ccb2_ref,
                  pi_ref, v_ref, vc_ref,
                  adjbf_scr, p0_scr, p1_scr, y_scr):
    ph = pl.program_id(0)
    i = pl.program_id(1)

    @pl.when(ph == 0)
    def _stream_layer0():
        blk = adj_ref[...]
        p0_scr[pl.ds(i * BLK, BLK), :] = _dot(blk, x_ref[...])
        adjbf_scr[pl.ds(i * BLK, BLK), :] = blk.astype(jnp.bfloat16)

    @pl.when((ph == 1) & (i == 0))
    def _layer_transition():
        hid = jax.nn.relu(_bn(_dot(p0_scr[...], g0w1_ref[...])))
        rep = _dot(hid, g0w2_ref[...])
        y_scr[...] = jax.nn.relu(_bn(rep)).astype(jnp.bfloat16)

    @pl.when(ph == 1)
    def _stream_layer1():
        p1_scr[pl.ds(i * BLK, BLK), :] = _dot(
            adjbf_scr[pl.ds(i * BLK, BLK), :], y_scr[...])

    @pl.when((ph == 1) & (i == NB - 1))
    def _epilogue():
        hid = jax.nn.relu(_bn(_dot(p1_scr[...], g1w1_ref[...])))
        rep = _dot(hid, g1w2_ref[...])
        h1 = jax.nn.relu(_bn(rep))

        hp = _dot(gp_ref[...], h1)

        r = jax.lax.broadcasted_iota(jnp.int32, (BC, 1), 0)
        bx = r // N_J
        bcols = jax.lax.broadcasted_iota(jnp.int32, (BC, B), 1)
        ohb = (bcols == bx).astype(jnp.float32)
        jcols = jax.lax.broadcasted_iota(jnp.int32, (BC, N_J), 1)
        ohj = (jcols == r % N_J).astype(jnp.float32)

        cand_rows = _dot(ohb, cand_ref[...].astype(jnp.float32), _HI)
        cc = (jnp.sum(cand_rows * ohj, axis=1, keepdims=True)
              .astype(jnp.int32) + bx * (N_J * N_M))
        mask_rows = _dot(ohb, mk_ref[...], _HI)
        mk = jnp.sum(mask_rows * ohj, axis=1, keepdims=True)

        cols = jax.lax.broadcasted_iota(jnp.int32, (BC, N), 1)
        oh = (cols == cc).astype(jnp.float32)
        cf = _dot(oh, h1, _HI)

        hpr = _dot(ohb, hp, _HI)

        aw1 = aw1_ref[...]
        t = jnp.tanh(_dot(cf, aw1[:HIDDEN, :])
                     + _dot(hpr, aw1[HIDDEN:, :]) + ab1_ref[...][None, :])
        s = _dot(t, aw2_ref[...]) + ab2_ref[...][None, :]
        s = jnp.where(mk != 0.0, -1e30, s)

        e = jnp.exp(s)
        den = jax.lax.dot_general(ohb, e, (((0,), (0,)), ((), ())),
                                  preferred_element_type=jnp.float32,
                                  precision=_HI)
        rden = _dot(ohb, den, _HI)
        pi_ref[...] = e / rden

        v_ref[...] = _dot(
            jnp.tanh(_dot(hp, crw1_ref[...]) + crb1_ref[...][None, :]),
            crw2_ref[...]) + crb2_ref[...][None, :]
        vc_ref[...] = _dot(
            jnp.tanh(_dot(hp, ccw1_ref[...]) + ccb1_ref[...][None, :]),
            ccw2_ref[...]) + ccb2_ref[...][None, :]


@jax.jit
def _run(x, graph_pool, adj, candidate, mask_f,
         g0w1, g0w2, g1w1, g1w2, aw1, ab1, aw2, ab2,
         crw1, crb1, crw2, crb2, ccw1, ccb1, ccw2, ccb2):
    res = lambda shp: pl.BlockSpec(shp, lambda p, i: (0,) * len(shp))
    pi_flat, v, v_c = pl.pallas_call(
        _fused_kernel,
        grid=(2, NB),
        in_specs=[
            pl.BlockSpec((BLK, N), lambda p, i: (i * (1 - p), 0)),
            res((N, INPUT_DIM)),
            res((B, N)),
            res((B, N_J)),
            res((B, N_J)),
            res((INPUT_DIM, HIDDEN)), res((HIDDEN, HIDDEN)),
            res((HIDDEN, HIDDEN)), res((HIDDEN, HIDDEN)),
            res((2 * HIDDEN, H_ACT)), res((H_ACT,)),
            res((H_ACT, 1)), res((1,)),
            res((HIDDEN, H_CRI)), res((H_CRI,)),
            res((H_CRI, 1)), res((1,)),
            res((HIDDEN, H_CRI)), res((H_CRI,)),
            res((H_CRI, 1)), res((1,)),
        ],
        out_specs=[res((BC, 1)), res((B, 1)), res((B, 1))],
        out_shape=[
            jax.ShapeDtypeStruct((BC, 1), jnp.float32),
            jax.ShapeDtypeStruct((B, 1), jnp.float32),
            jax.ShapeDtypeStruct((B, 1), jnp.float32),
        ],
        scratch_shapes=[
            pltpu.VMEM((N, N), jnp.bfloat16),
            pltpu.VMEM((N, INPUT_DIM), jnp.float32),
            pltpu.VMEM((N, HIDDEN), jnp.float32),
            pltpu.VMEM((N, HIDDEN), jnp.bfloat16),
        ],
        compiler_params=pltpu.CompilerParams(
            dimension_semantics=("arbitrary", "arbitrary")),
    )(adj, x, graph_pool, candidate, mask_f,
      g0w1, g0w2, g1w1, g1w2, aw1, ab1, aw2, ab2,
      crw1, crb1, crw2, crb2, ccw1, ccb1, ccw2, ccb2)
    return pi_flat.reshape(B, N_J, 1), v, v_c


def kernel(x, graph_pool, padded_nei, adj, candidate, mask,
           g0w1, g0b1, g0w2, g0b2, g1w1, g1b1, g1w2, g1b2,
           aw1, ab1, aw2, ab2, crw1, crb1, crw2, crb2,
           ccw1, ccb1, ccw2, ccb2):
    del padded_nei, g0b1, g0b2, g1b1, g1b2
    return _run(x, graph_pool, adj, candidate.astype(jnp.int32),
                mask.astype(jnp.float32),
                g0w1, g0w2, g1w1, g1w2, aw1, ab1, aw2, ab2,
                crw1, crb1, crw2, crb2, ccw1, ccb1, ccw2, ccb2)
